# baseline (speedup 1.0000x reference)
import functools

import jax
import jax.numpy as jnp
from jax import lax
from jax.experimental import pallas as pl
from jax.experimental.pallas import tpu as pltpu
from jax.experimental.pallas import tpu_sc as plsc

_B = 16384
_L = 50
_D = 64
_NC = 2
_NS = 16
_NW = _NC * _NS
_RPW = _B // _NW
_CH = 128
_NCH = _RPW // _CH
_NSL = _D // 16
_NBUF = 5


def _sc_body(tok, lat_i, lon_i, qtab, ltab, out,
             tok_r, tok_v, gbuf, acc, latb, lonb, lli, t0v, n0v,
             sem_g0, sem_g1, sem_g2, sem_g3, sem_g4, sem_aux, sem_out):
    wid = lax.axis_index("s") * _NC + lax.axis_index("c")
    base0 = wid * _RPW

    pltpu.sync_copy(qtab.at[0], t0v)

    gsems = (sem_g0, sem_g1, sem_g2, sem_g3, sem_g4)

    @pl.loop(0, _NCH)
    def _chunk(c):
        base = base0 + c * _CH

        pltpu.sync_copy(tok.at[pl.ds(base, _CH), :], tok_r)
        pltpu.sync_copy(lat_i.at[pl.ds(base, _CH)], lli.at[0])
        pltpu.sync_copy(lon_i.at[pl.ds(base, _CH)], lli.at[1])

        lanes = lax.broadcasted_iota(jnp.int32, (16,), 0)

        @pl.loop(0, _L)
        def _tr(j):
            jcol = jnp.full((16,), j, jnp.int32)
            for g in range(_CH // 16):
                v = plsc.load_gather(tok_r, [lanes + g * 16, jcol])
                tok_v[j, pl.ds(g * 16, 16)] = v

        cp_lat = pltpu.async_copy(ltab.at[lli.at[0]], latb, sem_aux)
        cp_lon = pltpu.async_copy(ltab.at[lli.at[1]], lonb, sem_aux)

        for b in range(_NBUF):
            pltpu.async_copy(qtab.at[tok_v.at[b]], gbuf.at[b], gsems[b])

        zeros = jnp.zeros((16,), jnp.float32)

        @pl.loop(0, _CH, unroll=4)
        def _zero(r):
            for k in range(_NSL):
                acc[r, pl.ds(k * 16, 16)] = zeros

        @pl.loop(0, _CH // 16)
        def _cnt(g):
            def jstep(j, cnt):
                t = tok_v[j, pl.ds(g * 16, 16)]
                return cnt + jnp.where(t == 0, 1, 0).astype(jnp.int32)

            cnt = pl.loop(0, _L, init_carry=jnp.zeros((16,), jnp.int32))(jstep)
            n0v[pl.ds(g * 16, 16)] = cnt

        @pl.loop(0, _L // _NBUF - 1)
        def _ring(jj):
            for b in range(_NBUF):
                j = jj * _NBUF + b
                pltpu.make_async_copy(
                    qtab.at[tok_v.at[0]], gbuf.at[b], gsems[b]).wait()

                @pl.loop(0, _CH, unroll=4)
                def _acc(r):
                    for k in range(_NSL):
                        sl = pl.ds(k * 16, 16)
                        plsc.addupdate(acc.at[r, sl], gbuf[b, r, sl])

                pltpu.async_copy(
                    qtab.at[tok_v.at[j + _NBUF]], gbuf.at[b], gsems[b])

        for b in range(_NBUF):
            pltpu.make_async_copy(
                qtab.at[tok_v.at[0]], gbuf.at[b], gsems[b]).wait()

            @pl.loop(0, _CH, unroll=4)
            def _acc_tail(r):
                for k in range(_NSL):
                    sl = pl.ds(k * 16, 16)
                    plsc.addupdate(acc.at[r, sl], gbuf[b, r, sl])

        @pl.loop(0, _CH // 16)
        def _fix(g):
            nf = n0v[pl.ds(g * 16, 16)].astype(jnp.float32)
            inv = 1.0 / jnp.maximum(jnp.float32(_L) - nf, 1.0)
            for e in range(16):
                r = g * 16 + e
                nfe = nf[e]
                inve = inv[e]
                for k in range(_NSL):
                    sl = pl.ds(k * 16, 16)
                    acc[r, sl] = (acc[r, sl] - nfe * t0v[sl]) * inve

        cp_lat.wait()
        cp_lon.wait()
        o1 = pltpu.async_copy(acc, out.at[pl.ds(base, _CH), pl.ds(0, _D)],
                              sem_out)
        o2 = pltpu.async_copy(latb, out.at[pl.ds(base, _CH), pl.ds(_D, _D)],
                              sem_out)
        o3 = pltpu.async_copy(lonb, out.at[pl.ds(base, _CH), pl.ds(2 * _D, _D)],
                              sem_out)
        o1.wait()
        o2.wait()
        o3.wait()


@jax.jit
def _run(tok, lat_i, lon_i, qtab, ltab):
    mesh = plsc.VectorSubcoreMesh(core_axis_name="c", subcore_axis_name="s")
    return pl.kernel(
        _sc_body,
        out_type=jax.ShapeDtypeStruct((_B, 3 * _D), jnp.float32),
        mesh=mesh,
        scratch_types=[
            pltpu.VMEM((_CH, _L), jnp.int32),
            pltpu.VMEM((_L, _CH), jnp.int32),
            pltpu.VMEM((_NBUF, _CH, _D), jnp.float32),
            pltpu.VMEM((_CH, _D), jnp.float32),
            pltpu.VMEM((_CH, _D), jnp.float32),
            pltpu.VMEM((_CH, _D), jnp.float32),
            pltpu.VMEM((2, _CH), jnp.int32),
            pltpu.VMEM((_D,), jnp.float32),
            pltpu.VMEM((_CH,), jnp.int32),
            pltpu.SemaphoreType.DMA,
            pltpu.SemaphoreType.DMA,
            pltpu.SemaphoreType.DMA,
            pltpu.SemaphoreType.DMA,
            pltpu.SemaphoreType.DMA,
            pltpu.SemaphoreType.DMA,
            pltpu.SemaphoreType.DMA,
        ],
        compiler_params=pltpu.CompilerParams(use_tc_tiling_on_sc=False,
                                             needs_layout_passes=False),
        name="query_model_sc",
    )(tok, lat_i, lon_i, qtab, ltab)


def kernel(query_tokens, wh_latitude, wh_longitude, query_table, lonlat_table):
    tok = query_tokens.astype(jnp.int32)
    lat_i = wh_latitude.astype(jnp.int32)
    lon_i = wh_longitude.astype(jnp.int32)
    return _run(tok, lat_i, lon_i,
                query_table.astype(jnp.float32),
                lonlat_table.astype(jnp.float32))


# speedup vs baseline: 1.1238x; 1.1238x over previous
import functools

import jax
import jax.numpy as jnp
from jax import lax
from jax.experimental import pallas as pl
from jax.experimental.pallas import tpu as pltpu
from jax.experimental.pallas import tpu_sc as plsc

_B = 16384
_L = 50
_D = 64
_NC = 2
_NS = 16
_NW = _NC * _NS
_RPW = _B // _NW
_CH = 128
_NCH = _RPW // _CH
_NSL = _D // 16
_NBUF = 5
_VQ = 10000
_VL = 1001


def _sc_body(tok, lat_i, lon_i, qtab, ltab, out,
             tok_v, gbuf, acc, latb, lonb, lli, t0v, n0v, sh_tab,
             sem_g0, sem_g1, sem_g2, sem_g3, sem_g4, sem_aux, sem_out):
    sid = lax.axis_index("s")
    wid = sid * _NC + lax.axis_index("c")
    base0 = wid * _RPW

    seg = _VQ // _NS
    pltpu.sync_copy(qtab.at[pl.ds(sid * seg, seg)],
                    sh_tab.at[pl.ds(sid * seg, seg)])
    plsc.subcore_barrier()

    pltpu.sync_copy(qtab.at[0], t0v)

    gsems = (sem_g0, sem_g1, sem_g2, sem_g3, sem_g4)

    @pl.loop(0, _NCH)
    def _chunk(c):
        base = base0 + c * _CH

        pltpu.sync_copy(tok.at[:, pl.ds(base, _CH)], tok_v)
        pltpu.sync_copy(lat_i.at[pl.ds(base, _CH)], lli.at[0])
        pltpu.sync_copy(lon_i.at[pl.ds(base, _CH)], lli.at[1])

        cp_lat = pltpu.async_copy(ltab.at[lli.at[0]], latb, sem_aux)
        cp_lon = pltpu.async_copy(ltab.at[lli.at[1]], lonb, sem_aux)

        for b in range(_NBUF):
            pltpu.async_copy(sh_tab.at[tok_v.at[b]], gbuf.at[b], gsems[b])

        zeros = jnp.zeros((16,), jnp.float32)

        @pl.loop(0, _CH, unroll=4)
        def _zero(r):
            for k in range(_NSL):
                acc[r, pl.ds(k * 16, 16)] = zeros

        @pl.loop(0, _CH // 16)
        def _cnt(g):
            def jstep(j, cnt):
                t = tok_v[j, pl.ds(g * 16, 16)]
                return cnt + jnp.where(t == 0, 1, 0).astype(jnp.int32)

            cnt = pl.loop(0, _L, init_carry=jnp.zeros((16,), jnp.int32))(jstep)
            n0v[pl.ds(g * 16, 16)] = cnt

        @pl.loop(0, _L // _NBUF - 1)
        def _ring(jj):
            for b in range(_NBUF):
                j = jj * _NBUF + b
                pltpu.make_async_copy(
                    sh_tab.at[tok_v.at[0]], gbuf.at[b], gsems[b]).wait()

                @pl.loop(0, _CH, unroll=4)
                def _acc(r):
                    for k in range(_NSL):
                        sl = pl.ds(k * 16, 16)
                        plsc.addupdate(acc.at[r, sl], gbuf[b, r, sl])

                pltpu.async_copy(
                    sh_tab.at[tok_v.at[j + _NBUF]], gbuf.at[b], gsems[b])

        for b in range(_NBUF):
            pltpu.make_async_copy(
                sh_tab.at[tok_v.at[0]], gbuf.at[b], gsems[b]).wait()

            @pl.loop(0, _CH, unroll=4)
            def _acc_tail(r):
                for k in range(_NSL):
                    sl = pl.ds(k * 16, 16)
                    plsc.addupdate(acc.at[r, sl], gbuf[b, r, sl])

        @pl.loop(0, _CH // 16)
        def _fix(g):
            nf = n0v[pl.ds(g * 16, 16)].astype(jnp.float32)
            inv = 1.0 / jnp.maximum(jnp.float32(_L) - nf, 1.0)
            for e in range(16):
                r = g * 16 + e
                nfe = nf[e]
                inve = inv[e]
                for k in range(_NSL):
                    sl = pl.ds(k * 16, 16)
                    acc[r, sl] = (acc[r, sl] - nfe * t0v[sl]) * inve

        cp_lat.wait()
        cp_lon.wait()
        o1 = pltpu.async_copy(acc, out.at[pl.ds(base, _CH), pl.ds(0, _D)],
                              sem_out)
        o2 = pltpu.async_copy(latb, out.at[pl.ds(base, _CH), pl.ds(_D, _D)],
                              sem_out)
        o3 = pltpu.async_copy(lonb, out.at[pl.ds(base, _CH), pl.ds(2 * _D, _D)],
                              sem_out)
        o1.wait()
        o2.wait()
        o3.wait()


@jax.jit
def _run(tok, lat_i, lon_i, qtab, ltab):
    mesh = plsc.VectorSubcoreMesh(core_axis_name="c", subcore_axis_name="s")
    return pl.kernel(
        _sc_body,
        out_type=jax.ShapeDtypeStruct((_B, 3 * _D), jnp.float32),
        mesh=mesh,
        scratch_types=[
            pltpu.VMEM((_L, _CH), jnp.int32),
            pltpu.VMEM((_NBUF, _CH, _D), jnp.float32),
            pltpu.VMEM((_CH, _D), jnp.float32),
            pltpu.VMEM((_CH, _D), jnp.float32),
            pltpu.VMEM((_CH, _D), jnp.float32),
            pltpu.VMEM((2, _CH), jnp.int32),
            pltpu.VMEM((_D,), jnp.float32),
            pltpu.VMEM((_CH,), jnp.int32),
            pltpu.VMEM_SHARED((_VQ, _D), jnp.float32),
            pltpu.SemaphoreType.DMA,
            pltpu.SemaphoreType.DMA,
            pltpu.SemaphoreType.DMA,
            pltpu.SemaphoreType.DMA,
            pltpu.SemaphoreType.DMA,
            pltpu.SemaphoreType.DMA,
            pltpu.SemaphoreType.DMA,
        ],
        compiler_params=pltpu.CompilerParams(use_tc_tiling_on_sc=False,
                                             needs_layout_passes=False),
        name="query_model_sc",
    )(tok, lat_i, lon_i, qtab, ltab)


def kernel(query_tokens, wh_latitude, wh_longitude, query_table, lonlat_table):
    tok = query_tokens.astype(jnp.int32).T
    lat_i = wh_latitude.astype(jnp.int32)
    lon_i = wh_longitude.astype(jnp.int32)
    return _run(tok, lat_i, lon_i,
                query_table.astype(jnp.float32),
                lonlat_table.astype(jnp.float32))
